# baseline (speedup 1.0000x reference)
import functools

import jax
import jax.numpy as jnp
from jax import lax
from jax.experimental import pallas as pl
from jax.experimental.pallas import tpu as pltpu
from jax.experimental.pallas import tpu_sc as plsc

B = 16
T = 4096
D = 1024
NSUB = 8


@functools.partial(
    pl.kernel,
    mesh=plsc.VectorSubcoreMesh(
        core_axis_name="c", subcore_axis_name="s", num_cores=1
    ),
    out_type=jax.ShapeDtypeStruct((B * NSUB, D // NSUB), jnp.float32),
    scratch_types=[
        pltpu.VMEM((B,), jnp.int32),
        pltpu.VMEM((B, D // NSUB), jnp.float32),
        pltpu.SemaphoreType.DMA,
    ],
)
def _last_relevant_sc(flat_hbm, len_hbm, out_hbm, len_v, rows_v, sem):
    sid = lax.axis_index("s")

    @pl.when(sid < NSUB)
    def _():
        pltpu.sync_copy(len_hbm, len_v)
        iota = lax.iota(jnp.int32, B)
        idx = (iota * T + (len_v[...] - 1)) * NSUB + sid
        pltpu.async_copy(flat_hbm.at[idx], rows_v, sem).wait()
        oidx = iota * NSUB + sid
        pltpu.async_copy(rows_v, out_hbm.at[oidx], sem).wait()


def kernel(outputs, tensor_len):
    flat = outputs.reshape(B * T * NSUB, D // NSUB)
    lens = tensor_len.reshape(-1).astype(jnp.int32)
    return _last_relevant_sc(flat, lens).reshape(B, D)


# speedup vs baseline: 13.8346x; 13.8346x over previous
import functools

import jax
import jax.numpy as jnp
from jax import lax
from jax.experimental import pallas as pl
from jax.experimental.pallas import tpu as pltpu
from jax.experimental.pallas import tpu_sc as plsc

B = 16
T = 4096
D = 1024


@functools.partial(
    pl.kernel,
    mesh=plsc.VectorSubcoreMesh(
        core_axis_name="c", subcore_axis_name="s", num_cores=1
    ),
    out_type=jax.ShapeDtypeStruct((B, D), jnp.float32),
    scratch_types=[
        pltpu.VMEM((B,), jnp.int32),
        pltpu.VMEM((B, D), jnp.float32),
        pltpu.SemaphoreType.DMA,
    ],
)
def _last_relevant_sc(flat_hbm, len_hbm, out_hbm, len_v, rows_v, sem):
    sid = lax.axis_index("s")

    @pl.when(sid == 0)
    def _():
        pltpu.sync_copy(len_hbm, len_v)
        idx = lax.iota(jnp.int32, B) * T + (len_v[...] - 1)
        pltpu.async_copy(flat_hbm.at[idx], rows_v, sem).wait()
        pltpu.sync_copy(rows_v, out_hbm)


def kernel(outputs, tensor_len):
    flat = outputs.reshape(B * T, D)
    lens = tensor_len.reshape(-1).astype(jnp.int32)
    return _last_relevant_sc(flat, lens)


# speedup vs baseline: 14.3641x; 1.0383x over previous
import functools

import jax
import jax.numpy as jnp
from jax import lax
from jax.experimental import pallas as pl
from jax.experimental.pallas import tpu as pltpu
from jax.experimental.pallas import tpu_sc as plsc

B = 16
T = 4096
D = 1024


@functools.partial(
    pl.kernel,
    mesh=plsc.ScalarSubcoreMesh(axis_name="c", num_cores=1),
    out_type=jax.ShapeDtypeStruct((B, D), jnp.float32),
    scratch_types=[
        pltpu.SMEM((B,), jnp.int32),
        pltpu.SemaphoreType.DMA,
    ],
)
def _last_relevant_sc(flat_hbm, len_hbm, out_hbm, len_s, sem):
    pltpu.sync_copy(len_hbm, len_s)
    copies = []
    for b in range(B):
        idx = b * T + (len_s[b] - 1)
        copies.append(
            pltpu.async_copy(
                flat_hbm.at[pl.ds(idx, 1)], out_hbm.at[pl.ds(b, 1)], sem
            )
        )
    for c in copies:
        c.wait()


def kernel(outputs, tensor_len):
    flat = outputs.reshape(B * T, D)
    lens = tensor_len.reshape(-1).astype(jnp.int32)
    return _last_relevant_sc(flat, lens)
